# Initial kernel scaffold; baseline (speedup 1.0000x reference)
import jax
import jax.numpy as jnp
from jax.experimental import pallas as pl


def kernel(inp, emb_0, emb_1, emb_2, proj_0, proj_1, proj_2):
    raise NotImplementedError("write your pallas kernel here")


# speedup vs baseline: 19.7153x; 19.7153x over previous
import functools

import jax
import jax.numpy as jnp
from jax import lax
from jax.experimental import pallas as pl
from jax.experimental.pallas import tpu as pltpu
from jax.experimental.pallas import tpu_sc as plsc

_CUTS = (0, 20000, 100000, 1000000)
_D = 128
_SCALE = float(_D) ** 0.5

_NC, _NS = 2, 16
_NW = _NC * _NS
_V = _CUTS[3]
_BR = 2000

_B = 4096 * 200
_CB = 128
_BPW = _B // _NW
_NCH = _BPW // _CB


def _proj_kernel(emb_ref, proj_ref, out_ref):
    out_ref[...] = lax.dot_general(
        emb_ref[...], proj_ref[...],
        (((1,), (1,)), ((), ())),
        preferred_element_type=jnp.float32) * _SCALE


def _proj_kernel_aliased(tab_ref, emb_ref, proj_ref, out_ref):
    del tab_ref
    _proj_kernel(emb_ref, proj_ref, out_ref)


def _build_table(emb_0, emb_1, emb_2, proj_0, proj_1, proj_2):
    out_shape = jax.ShapeDtypeStruct((_V, _D), jnp.float32)
    table = pl.pallas_call(
        _proj_kernel,
        grid=(_CUTS[1] // _BR,),
        in_specs=[pl.BlockSpec((_BR, 128), lambda i: (i, 0)),
                  pl.BlockSpec((128, 128), lambda i: (0, 0))],
        out_specs=pl.BlockSpec((_BR, _D), lambda i: (i, 0)),
        out_shape=out_shape,
    )(emb_0, proj_0)
    for emb, proj, lo, hi in ((emb_1, proj_1, _CUTS[1], _CUTS[2]),
                              (emb_2, proj_2, _CUTS[2], _CUTS[3])):
        off = lo // _BR
        d_in = emb.shape[1]
        table = pl.pallas_call(
            _proj_kernel_aliased,
            grid=((hi - lo) // _BR,),
            in_specs=[pl.BlockSpec(memory_space=pl.ANY),
                      pl.BlockSpec((_BR, d_in), lambda i: (i, 0)),
                      pl.BlockSpec((128, d_in), lambda i: (0, 0))],
            out_specs=pl.BlockSpec((_BR, _D),
                                   lambda i, _off=off: (i + _off, 0)),
            out_shape=out_shape,
            input_output_aliases={0: 0},
        )(table, emb, proj)
    return table


def _sc_lookup_body(table_hbm, idx_hbm, out_hbm, idx_v, rows_v, sem):
    wid = lax.axis_index("s") * _NC + lax.axis_index("c")
    base = wid * _BPW
    pltpu.sync_copy(idx_hbm.at[pl.ds(wid * _NCH, _NCH)], idx_v)

    def body(g, carry):
        pltpu.async_copy(table_hbm.at[idx_v.at[g]], rows_v, sem).wait()
        pltpu.sync_copy(rows_v, out_hbm.at[pl.ds(base + g * _CB, _CB)])
        return carry

    lax.fori_loop(0, _NCH, body, 0)


_sc_lookup = pl.kernel(
    _sc_lookup_body,
    out_type=jax.ShapeDtypeStruct((_B, _D), jnp.float32),
    mesh=plsc.VectorSubcoreMesh(core_axis_name="c", subcore_axis_name="s"),
    scratch_types=[
        pltpu.VMEM((_NCH, _CB), jnp.int32),
        pltpu.VMEM((_CB, _D), jnp.float32),
        pltpu.SemaphoreType.DMA,
    ],
)


def kernel(inp, emb_0, emb_1, emb_2, proj_0, proj_1, proj_2):
    table = _build_table(emb_0, emb_1, emb_2, proj_0, proj_1, proj_2)
    idx2d = inp.reshape(_B // _CB, _CB)
    out = _sc_lookup(table, idx2d)
    return out.reshape(inp.shape[0], inp.shape[1], _D)


# speedup vs baseline: 22.2149x; 1.1268x over previous
import functools

import jax
import jax.numpy as jnp
from jax import lax
from jax.experimental import pallas as pl
from jax.experimental.pallas import tpu as pltpu
from jax.experimental.pallas import tpu_sc as plsc

_CUTS = (0, 20000, 100000, 1000000)
_D = 128
_SCALE = float(_D) ** 0.5

_NC, _NS = 2, 16
_NW = _NC * _NS
_V = _CUTS[3]
_BR = 2000

_B = 4096 * 200
_CB = 128
_BPW = _B // _NW
_NCH = _BPW // _CB


def _proj_kernel(emb_ref, proj_ref, out_ref):
    out_ref[...] = lax.dot_general(
        emb_ref[...], proj_ref[...],
        (((1,), (1,)), ((), ())),
        preferred_element_type=jnp.float32) * _SCALE


def _proj_kernel_aliased(tab_ref, emb_ref, proj_ref, out_ref):
    del tab_ref
    _proj_kernel(emb_ref, proj_ref, out_ref)


def _build_table(emb_0, emb_1, emb_2, proj_0, proj_1, proj_2):
    out_shape = jax.ShapeDtypeStruct((_V, _D), jnp.float32)
    table = pl.pallas_call(
        _proj_kernel,
        grid=(_CUTS[1] // _BR,),
        in_specs=[pl.BlockSpec((_BR, 128), lambda i: (i, 0)),
                  pl.BlockSpec((128, 128), lambda i: (0, 0))],
        out_specs=pl.BlockSpec((_BR, _D), lambda i: (i, 0)),
        out_shape=out_shape,
    )(emb_0, proj_0)
    for emb, proj, lo, hi in ((emb_1, proj_1, _CUTS[1], _CUTS[2]),
                              (emb_2, proj_2, _CUTS[2], _CUTS[3])):
        off = lo // _BR
        d_in = emb.shape[1]
        table = pl.pallas_call(
            _proj_kernel_aliased,
            grid=((hi - lo) // _BR,),
            in_specs=[pl.BlockSpec(memory_space=pl.ANY),
                      pl.BlockSpec((_BR, d_in), lambda i: (i, 0)),
                      pl.BlockSpec((128, d_in), lambda i: (0, 0))],
            out_specs=pl.BlockSpec((_BR, _D),
                                   lambda i, _off=off: (i + _off, 0)),
            out_shape=out_shape,
            input_output_aliases={0: 0},
        )(table, emb, proj)
    return table


def _sc_lookup_body(table_hbm, idx_hbm, out_hbm, idx_v, rows_v, sem0, sem1):
    wid = lax.axis_index("s") * _NC + lax.axis_index("c")
    base = wid * _BPW
    pltpu.sync_copy(idx_hbm.at[pl.ds(wid * _NCH, _NCH)], idx_v)

    def start(g, slot, sem):
        pltpu.async_copy(table_hbm.at[idx_v.at[g]], rows_v.at[slot], sem)

    def drain(slot, sem):
        pltpu.make_async_copy(table_hbm.at[idx_v.at[0]], rows_v.at[slot],
                              sem).wait()

    def put(g, slot):
        pltpu.sync_copy(rows_v.at[slot], out_hbm.at[pl.ds(base + g * _CB, _CB)])

    start(0, 0, sem0)

    def body(p, carry):
        g0 = 2 * p
        start(g0 + 1, 1, sem1)
        drain(0, sem0)
        put(g0, 0)

        @pl.when(g0 + 2 < _NCH)
        def _():
            start(g0 + 2, 0, sem0)

        drain(1, sem1)
        put(g0 + 1, 1)
        return carry

    lax.fori_loop(0, _NCH // 2, body, 0)


_sc_lookup = pl.kernel(
    _sc_lookup_body,
    out_type=jax.ShapeDtypeStruct((_B, _D), jnp.float32),
    mesh=plsc.VectorSubcoreMesh(core_axis_name="c", subcore_axis_name="s"),
    scratch_types=[
        pltpu.VMEM((_NCH, _CB), jnp.int32),
        pltpu.VMEM((2, _CB, _D), jnp.float32),
        pltpu.SemaphoreType.DMA,
        pltpu.SemaphoreType.DMA,
    ],
)


def kernel(inp, emb_0, emb_1, emb_2, proj_0, proj_1, proj_2):
    table = _build_table(emb_0, emb_1, emb_2, proj_0, proj_1, proj_2)
    idx2d = inp.reshape(_B // _CB, _CB)
    out = _sc_lookup(table, idx2d)
    return out.reshape(inp.shape[0], inp.shape[1], _D)


# speedup vs baseline: 27.5041x; 1.2381x over previous
import functools

import jax
import jax.numpy as jnp
from jax import lax
from jax.experimental import pallas as pl
from jax.experimental.pallas import tpu as pltpu
from jax.experimental.pallas import tpu_sc as plsc

_CUTS = (0, 20000, 100000, 1000000)
_D = 128
_SCALE = float(_D) ** 0.5

_NC, _NS = 2, 16
_NW = _NC * _NS
_V = _CUTS[3]
_BR = 2000

_B = 4096 * 200
_CB = 128
_BPW = _B // _NW
_NCH = _BPW // _CB


def _proj_kernel(emb_ref, proj_ref, out_ref):
    out_ref[...] = lax.dot_general(
        emb_ref[...], proj_ref[...] * _SCALE,
        (((1,), (1,)), ((), ())),
        preferred_element_type=jnp.float32)


def _proj_kernel_aliased(tab_ref, emb_ref, proj_ref, out_ref):
    del tab_ref
    _proj_kernel(emb_ref, proj_ref, out_ref)


def _build_table(emb_0, emb_1, emb_2, proj_0, proj_1, proj_2):
    out_shape = jax.ShapeDtypeStruct((_V, _D), jnp.float32)
    br0 = 4000
    table = pl.pallas_call(
        _proj_kernel,
        grid=(_CUTS[1] // br0,),
        in_specs=[pl.BlockSpec((br0, 128), lambda i: (i, 0)),
                  pl.BlockSpec((128, 128), lambda i: (0, 0))],
        out_specs=pl.BlockSpec((br0, _D), lambda i: (i, 0)),
        out_shape=out_shape,
    )(emb_0, proj_0)
    for emb, proj, lo, hi, br in (
            (emb_1, proj_1, _CUTS[1], _CUTS[2], 4000),
            (emb_2, proj_2, _CUTS[2], _CUTS[3], 20000)):
        off = lo // br
        d_in = emb.shape[1]
        table = pl.pallas_call(
            _proj_kernel_aliased,
            grid=((hi - lo) // br,),
            in_specs=[pl.BlockSpec(memory_space=pl.ANY),
                      pl.BlockSpec((br, d_in), lambda i: (i, 0)),
                      pl.BlockSpec((128, d_in), lambda i: (0, 0))],
            out_specs=pl.BlockSpec((br, _D),
                                   lambda i, _off=off: (i + _off, 0)),
            out_shape=out_shape,
            input_output_aliases={0: 0},
        )(table, emb, proj)
    return table


def _sc_lookup_body(table_hbm, idx_hbm, out_hbm, idx_v, rows_v, sem0, sem1):
    wid = lax.axis_index("s") * _NC + lax.axis_index("c")
    base = wid * _BPW
    pltpu.sync_copy(idx_hbm.at[pl.ds(wid * _NCH, _NCH)], idx_v)

    def start(g, slot, sem):
        pltpu.async_copy(table_hbm.at[idx_v.at[g]], rows_v.at[slot], sem)

    def drain(slot, sem):
        pltpu.make_async_copy(table_hbm.at[idx_v.at[0]], rows_v.at[slot],
                              sem).wait()

    def put(g, slot):
        pltpu.sync_copy(rows_v.at[slot], out_hbm.at[pl.ds(base + g * _CB, _CB)])

    start(0, 0, sem0)

    def body(p, carry):
        g0 = 2 * p
        start(g0 + 1, 1, sem1)
        drain(0, sem0)
        put(g0, 0)

        @pl.when(g0 + 2 < _NCH)
        def _():
            start(g0 + 2, 0, sem0)

        drain(1, sem1)
        put(g0 + 1, 1)
        return carry

    lax.fori_loop(0, _NCH // 2, body, 0)


_sc_lookup = pl.kernel(
    _sc_lookup_body,
    out_type=jax.ShapeDtypeStruct((_B, _D), jnp.float32),
    mesh=plsc.VectorSubcoreMesh(core_axis_name="c", subcore_axis_name="s"),
    scratch_types=[
        pltpu.VMEM((_NCH, _CB), jnp.int32),
        pltpu.VMEM((2, _CB, _D), jnp.float32),
        pltpu.SemaphoreType.DMA,
        pltpu.SemaphoreType.DMA,
    ],
)


def kernel(inp, emb_0, emb_1, emb_2, proj_0, proj_1, proj_2):
    table = _build_table(emb_0, emb_1, emb_2, proj_0, proj_1, proj_2)
    idx2d = inp.reshape(_B // _CB, _CB)
    out = _sc_lookup(table, idx2d)
    return out.reshape(inp.shape[0], inp.shape[1], _D)
